# Initial kernel scaffold; baseline (speedup 1.0000x reference)
import jax
import jax.numpy as jnp
from jax.experimental import pallas as pl


def kernel(logits, beam_scores, finished):
    raise NotImplementedError("write your pallas kernel here")


# speedup vs baseline: 46.3649x; 46.3649x over previous
import functools

import jax
import jax.numpy as jnp
from jax.experimental import pallas as pl

BEAM_WIDTH = 8
EOS_ID = 2
BATCH = 64
VOCAB = 100000
K = 8
ROWS_PER_BLOCK = 8
NEG_INF = float("-inf")


def _topk_rows_kernel(x_ref, v_ref, i_ref, lse_ref):
    x = x_ref[...]
    r, v = x.shape
    m = jnp.max(x, axis=1, keepdims=True)
    s = jnp.sum(jnp.exp(x - m), axis=1, keepdims=True)
    lse = m + jnp.log(s)
    lse_ref[...] = jnp.broadcast_to(lse, (r, K))

    cols = jax.lax.broadcasted_iota(jnp.int32, (r, v), 1)
    y = x
    vals = []
    idxs = []
    for _ in range(K):
        mj = jnp.max(y, axis=1, keepdims=True)
        is_m = y == mj
        ij = jnp.min(jnp.where(is_m, cols, v), axis=1, keepdims=True)
        vals.append(mj)
        idxs.append(ij)
        y = jnp.where(cols == ij, NEG_INF, y)
    v_ref[...] = jnp.concatenate(vals, axis=1)
    i_ref[...] = jnp.concatenate(idxs, axis=1)


def _merge_kernel(v_ref, i_ref, lse_ref, bs_ref, fin_ref,
                  score_ref, tok_ref, par_ref, nf_ref):
    v = v_ref[...]
    ids = i_ref[...]
    lse = lse_ref[...]
    bs = bs_ref[...]
    fin = fin_ref[...]

    n = BEAM_WIDTH * K
    cols = jax.lax.broadcasted_iota(jnp.int32, (BATCH, n), 1)
    j_in_beam = cols - (cols // K) * K
    live_score = bs + v - lse
    fin_score = jnp.where(j_in_beam == 0, bs, NEG_INF)
    cand = jnp.where(fin == 1, fin_score, live_score)
    tok = jnp.where(fin == 1, EOS_ID, ids)

    scores = []
    toks = []
    pars = []
    nfs = []
    y = cand
    for _ in range(BEAM_WIDTH):
        mj = jnp.max(y, axis=1, keepdims=True)
        ij = jnp.min(jnp.where(y == mj, cols, n), axis=1, keepdims=True)
        sel = cols == ij
        tj = jnp.sum(jnp.where(sel, tok, 0), axis=1, keepdims=True)
        fj = jnp.sum(jnp.where(sel, fin, 0), axis=1, keepdims=True)
        scores.append(mj)
        toks.append(tj)
        pars.append(ij // K)
        nfs.append(jnp.where((fj == 1) | (tj == EOS_ID), 1, 0))
        y = jnp.where(sel, NEG_INF, y)
    score_ref[...] = jnp.concatenate(scores, axis=1)
    tok_ref[...] = jnp.concatenate(toks, axis=1)
    par_ref[...] = jnp.concatenate(pars, axis=1)
    nf_ref[...] = jnp.concatenate(nfs, axis=1)


@jax.jit
def kernel(logits, beam_scores, finished):
    rows = BATCH * BEAM_WIDTH
    grid = rows // ROWS_PER_BLOCK
    v8, i8, lse8 = pl.pallas_call(
        _topk_rows_kernel,
        grid=(grid,),
        in_specs=[pl.BlockSpec((ROWS_PER_BLOCK, VOCAB), lambda i: (i, 0))],
        out_specs=[
            pl.BlockSpec((ROWS_PER_BLOCK, K), lambda i: (i, 0)),
            pl.BlockSpec((ROWS_PER_BLOCK, K), lambda i: (i, 0)),
            pl.BlockSpec((ROWS_PER_BLOCK, K), lambda i: (i, 0)),
        ],
        out_shape=[
            jax.ShapeDtypeStruct((rows, K), jnp.float32),
            jax.ShapeDtypeStruct((rows, K), jnp.int32),
            jax.ShapeDtypeStruct((rows, K), jnp.float32),
        ],
    )(logits)

    n = BEAM_WIDTH * K
    v64 = v8.reshape(BATCH, n)
    i64 = i8.reshape(BATCH, n)
    lse64 = lse8.reshape(BATCH, n)
    bs64 = jnp.repeat(beam_scores, K, axis=1)
    fin64 = jnp.repeat(finished.astype(jnp.int32), K, axis=1)

    scores, toks, pars, nf = pl.pallas_call(
        _merge_kernel,
        out_shape=[
            jax.ShapeDtypeStruct((BATCH, BEAM_WIDTH), jnp.float32),
            jax.ShapeDtypeStruct((BATCH, BEAM_WIDTH), jnp.int32),
            jax.ShapeDtypeStruct((BATCH, BEAM_WIDTH), jnp.int32),
            jax.ShapeDtypeStruct((BATCH, BEAM_WIDTH), jnp.int32),
        ],
    )(v64, i64, lse64, bs64, fin64)
    return scores, toks, pars, nf.astype(bool)
